# Initial kernel scaffold; baseline (speedup 1.0000x reference)
import jax
import jax.numpy as jnp
from jax.experimental import pallas as pl


def kernel(x, edge_index, meta_edge_index, meta_x, W_lin, b_lin, W_mlin, b_mlin, W_conv0, b_conv0, W_conv1, b_conv1, W_mg, b_mg, W_cls, b_cls):
    raise NotImplementedError("write your pallas kernel here")


# speedup vs baseline: 8.4153x; 8.4153x over previous
import functools

import jax
import jax.numpy as jnp
from jax import lax
from jax.experimental import pallas as pl
from jax.experimental.pallas import tpu as pltpu
from jax.experimental.pallas import tpu_sc as plsc

N = 10000
M = 3
D = 128
H = 128
C = 40
E = 320000
EM = 20000
P = 10240
NC, NS = 2, 16
NW = NC * NS
K = 80
EP = 327680
CM = EP // (NW * K)
EMP = 20480
CMM = EMP // (NW * K)
RPS = P // NS
ALPHA = 0.2
NEG = -1e30
GW = 8


def _lrelu(v):
    return jnp.where(v >= 0, v, ALPHA * v)


def _sc_mesh():
    return plsc.VectorSubcoreMesh(core_axis_name="c", subcore_axis_name="s")


@functools.partial(
    pl.kernel,
    out_type=(
        jax.ShapeDtypeStruct((NC, P, GW), jnp.float32),
        jax.ShapeDtypeStruct((NC, P, GW), jnp.float32),
    ),
    mesh=_sc_mesh(),
    scratch_types=[
        pltpu.VMEM_SHARED((P, GW), jnp.float32),
        pltpu.VMEM_SHARED((P, GW), jnp.float32),
        pltpu.VMEM((CM, K), jnp.int32),
        pltpu.VMEM((CMM, K), jnp.int32),
        pltpu.VMEM((K, GW), jnp.float32),
    ],
)
def _sc_degrees(dst_hbm, mdst_hbm, zero8_hbm, ones8_hbm,
                outd_hbm, outmd_hbm, accd, accmd, dst_v, mdst_v, ones_v):
    cid = lax.axis_index("c")
    sid = lax.axis_index("s")
    wid = cid * NS + sid
    row0 = sid * RPS
    pltpu.sync_copy(zero8_hbm, accd.at[pl.ds(row0, RPS)])
    pltpu.sync_copy(zero8_hbm, accmd.at[pl.ds(row0, RPS)])
    pltpu.sync_copy(ones8_hbm, ones_v)
    pltpu.sync_copy(dst_hbm.at[pl.ds(wid * CM, CM)], dst_v)
    pltpu.sync_copy(mdst_hbm.at[pl.ds(wid * CMM, CMM)], mdst_v)
    plsc.subcore_barrier()

    def body(c, carry):
        pltpu.sync_copy(ones_v, accd.at[dst_v.at[c]], add=True)
        return carry

    lax.fori_loop(0, CM, body, 0)

    def mbody(c, carry):
        pltpu.sync_copy(ones_v, accmd.at[mdst_v.at[c]], add=True)
        return carry

    lax.fori_loop(0, CMM, mbody, 0)
    plsc.subcore_barrier()
    pltpu.sync_copy(accd.at[pl.ds(row0, RPS)], outd_hbm.at[cid, pl.ds(row0, RPS)])
    pltpu.sync_copy(accmd.at[pl.ds(row0, RPS)], outmd_hbm.at[cid, pl.ds(row0, RPS)])


def _make_agg(cpw):
    ib = min(16, cpw)
    nb = cpw // ib
    assert cpw % ib == 0 and ib % 2 == 0

    @functools.partial(
        pl.kernel,
        out_type=jax.ShapeDtypeStruct((NC, P, H), jnp.float32),
        mesh=_sc_mesh(),
        scratch_types=[
            pltpu.VMEM_SHARED((P, H), jnp.float32),
            pltpu.VMEM((ib, K), jnp.int32),
            pltpu.VMEM((ib, K), jnp.int32),
            pltpu.VMEM((K, H), jnp.float32),
            pltpu.VMEM((K, H), jnp.float32),
            pltpu.SemaphoreType.DMA,
            pltpu.SemaphoreType.DMA,
        ],
    )
    def agg(z_hbm, src_hbm, dst_hbm, zero_hbm, out_hbm,
            acc, src_v, dst_v, rows_a, rows_b, sem_a, sem_b):
        cid = lax.axis_index("c")
        sid = lax.axis_index("s")
        wid = cid * NS + sid
        row0 = sid * RPS
        pltpu.sync_copy(zero_hbm, acc.at[pl.ds(row0, RPS)])
        plsc.subcore_barrier()

        def pair(c0, c1):
            da = pltpu.async_copy(z_hbm.at[src_v.at[c0]], rows_a, sem_a)
            db = pltpu.async_copy(z_hbm.at[src_v.at[c1]], rows_b, sem_b)
            da.wait()
            pltpu.sync_copy(rows_a, acc.at[dst_v.at[c0]], add=True)
            db.wait()
            pltpu.sync_copy(rows_b, acc.at[dst_v.at[c1]], add=True)

        def block(b, carry):
            pltpu.sync_copy(src_hbm.at[pl.ds(wid * cpw + b * ib, ib)], src_v)
            pltpu.sync_copy(dst_hbm.at[pl.ds(wid * cpw + b * ib, ib)], dst_v)

            def body(i, carry2):
                pair(2 * i, 2 * i + 1)
                return carry2

            return lax.fori_loop(0, ib // 2, body, carry)

        lax.fori_loop(0, nb, block, 0)
        plsc.subcore_barrier()
        pltpu.sync_copy(acc.at[pl.ds(row0, RPS)],
                        out_hbm.at[cid, pl.ds(row0, RPS)])

    return agg


_agg_main = _make_agg(CM)
_agg_meta = _make_agg(CMM)


def _entry_body(x_ref, wl_ref, bl_ref, mx_ref, wm_ref, bm_ref, o1_ref, o2_ref):
    o1_ref[...] = _lrelu(
        jnp.dot(x_ref[...], wl_ref[...], preferred_element_type=jnp.float32)
        + bl_ref[...])
    o2_ref[...] = _lrelu(
        jnp.dot(mx_ref[...], wm_ref[...], preferred_element_type=jnp.float32)
        + bm_ref[...])


def _dinv(deg):
    return lax.rsqrt(deg[0, :, 0:1] + deg[1, :, 0:1] + 1.0)


def _scale_body(h_ref, w_ref, deg_ref, z_ref):
    z_ref[...] = _dinv(deg_ref[...]) * jnp.dot(
        h_ref[...], w_ref[...], preferred_element_type=jnp.float32)


def _mid_body(agg_ref, z_ref, deg_ref, b_ref, w_ref, o_ref):
    dinv = _dinv(deg_ref[...])
    h = _lrelu(dinv * (agg_ref[0] + agg_ref[1] + z_ref[...]) + b_ref[...])
    o_ref[...] = dinv * jnp.dot(h, w_ref[...],
                                preferred_element_type=jnp.float32)


def _meta_body(agg_ref, z_ref, deg_ref, b_ref, mx_ref, w_ref, mdeg_ref, o_ref):
    dinv = _dinv(deg_ref[...])
    h = _lrelu(dinv * (agg_ref[0] + agg_ref[1] + z_ref[...]) + b_ref[...])
    h = jnp.concatenate([h[:N], mx_ref[...], h[N + 8:]], axis=0)
    o_ref[...] = _dinv(mdeg_ref[...]) * jnp.dot(
        h, w_ref[...], preferred_element_type=jnp.float32)


def _final_body(agg_ref, z_ref, mdeg_ref, b_ref, wc_ref, bc_ref, o_ref):
    dinv = _dinv(mdeg_ref[...])
    xm = _lrelu(dinv * (agg_ref[0] + agg_ref[1] + z_ref[...]) + b_ref[...])
    logits = jnp.dot(xm, wc_ref[...],
                     preferred_element_type=jnp.float32) + bc_ref[...]
    mx = jnp.max(logits, axis=1, keepdims=True)
    lse = jnp.log(jnp.sum(jnp.exp(logits - mx), axis=1, keepdims=True))
    o_ref[...] = logits - mx - lse


def _tc(body, out_shape, *args):
    return pl.pallas_call(body, out_shape=out_shape)(*args)


def kernel(x, edge_index, meta_edge_index, meta_x, W_lin, b_lin, W_mlin,
           b_mlin, W_conv0, b_conv0, W_conv1, b_conv1, W_mg, b_mg,
           W_cls, b_cls):
    f32 = jnp.float32
    x_pad = jnp.zeros((P, D), f32).at[:N].set(x)
    mx_pad = jnp.zeros((8, D), f32).at[:M].set(meta_x)
    src = jnp.full((EP,), P - 1, jnp.int32).at[:E].set(
        edge_index[0].astype(jnp.int32)).reshape(EP // K, K)
    dst = jnp.full((EP,), P - 1, jnp.int32).at[:E].set(
        edge_index[1].astype(jnp.int32)).reshape(EP // K, K)
    msrc = jnp.full((EMP,), P - 1, jnp.int32).at[:EM].set(
        meta_edge_index[0].astype(jnp.int32)).reshape(EMP // K, K)
    mdst = jnp.full((EMP,), P - 1, jnp.int32).at[:EM].set(
        meta_edge_index[1].astype(jnp.int32)).reshape(EMP // K, K)
    zero128 = jnp.zeros((RPS, H), f32)
    zero8 = jnp.zeros((RPS, GW), f32)
    ones8 = jnp.ones((K, GW), f32)
    wcls_pad = jnp.zeros((H, 128), f32).at[:, :C].set(W_cls)
    bcls_pad = jnp.full((1, 128), NEG, f32).at[0, :C].set(b_cls)

    degp, mdegp = _sc_degrees(dst, mdst, zero8, ones8)

    sds = jax.ShapeDtypeStruct
    x1, mx1 = _tc(_entry_body, (sds((P, H), f32), sds((8, H), f32)),
                  x_pad, W_lin, b_lin.reshape(1, H),
                  mx_pad, W_mlin, b_mlin.reshape(1, H))

    z0 = _tc(_scale_body, sds((P, H), f32), x1, W_conv0, degp)
    agg0 = _agg_main(z0, src, dst, zero128)
    z1 = _tc(_mid_body, sds((P, H), f32),
             agg0, z0, degp, b_conv0.reshape(1, H), W_conv1)
    agg1 = _agg_main(z1, src, dst, zero128)
    z2 = _tc(_meta_body, sds((P, H), f32),
             agg1, z1, degp, b_conv1.reshape(1, H), mx1, W_mg, mdegp)
    magg = _agg_meta(z2, msrc, mdst, zero128)
    outp = _tc(_final_body, sds((P, 128), f32),
               magg, z2, mdegp, b_mg.reshape(1, H), wcls_pad, bcls_pad)
    return outp[:N + M, :C]


# speedup vs baseline: 8.5540x; 1.0165x over previous
import functools

import jax
import jax.numpy as jnp
from jax import lax
from jax.experimental import pallas as pl
from jax.experimental.pallas import tpu as pltpu
from jax.experimental.pallas import tpu_sc as plsc

N = 10000
M = 3
D = 128
H = 128
C = 40
E = 320000
EM = 20000
P = 10240
NC, NS = 2, 16
NW = NC * NS
K = 80
EP = 327680
CM = EP // (NW * K)
EMP = 20480
CMM = EMP // (NW * K)
RPS = P // NS
ALPHA = 0.2
NEG = -1e30
GW = 8


def _lrelu(v):
    return jnp.where(v >= 0, v, ALPHA * v)


def _sc_mesh():
    return plsc.VectorSubcoreMesh(core_axis_name="c", subcore_axis_name="s")


@functools.partial(
    pl.kernel,
    out_type=(
        jax.ShapeDtypeStruct((NC, P, GW), jnp.float32),
        jax.ShapeDtypeStruct((NC, P, GW), jnp.float32),
    ),
    mesh=_sc_mesh(),
    scratch_types=[
        pltpu.VMEM_SHARED((P, GW), jnp.float32),
        pltpu.VMEM_SHARED((P, GW), jnp.float32),
        pltpu.VMEM((CM, K), jnp.int32),
        pltpu.VMEM((CMM, K), jnp.int32),
        pltpu.VMEM((K, GW), jnp.float32),
    ],
)
def _sc_degrees(dst_hbm, mdst_hbm, zero8_hbm, ones8_hbm,
                outd_hbm, outmd_hbm, accd, accmd, dst_v, mdst_v, ones_v):
    cid = lax.axis_index("c")
    sid = lax.axis_index("s")
    wid = cid * NS + sid
    row0 = sid * RPS
    pltpu.sync_copy(zero8_hbm, accd.at[pl.ds(row0, RPS)])
    pltpu.sync_copy(zero8_hbm, accmd.at[pl.ds(row0, RPS)])
    pltpu.sync_copy(ones8_hbm, ones_v)
    pltpu.sync_copy(dst_hbm.at[pl.ds(wid * CM, CM)], dst_v)
    pltpu.sync_copy(mdst_hbm.at[pl.ds(wid * CMM, CMM)], mdst_v)
    plsc.subcore_barrier()

    def body(c, carry):
        pltpu.sync_copy(ones_v, accd.at[dst_v.at[c]], add=True)
        return carry

    lax.fori_loop(0, CM, body, 0)

    def mbody(c, carry):
        pltpu.sync_copy(ones_v, accmd.at[mdst_v.at[c]], add=True)
        return carry

    lax.fori_loop(0, CMM, mbody, 0)
    plsc.subcore_barrier()
    pltpu.sync_copy(accd.at[pl.ds(row0, RPS)], outd_hbm.at[cid, pl.ds(row0, RPS)])
    pltpu.sync_copy(accmd.at[pl.ds(row0, RPS)], outmd_hbm.at[cid, pl.ds(row0, RPS)])


def _make_agg(cpw):
    ib = min(16, cpw)
    nb = cpw // ib
    nbuf = 4 if cpw % 4 == 0 else 2
    assert cpw % ib == 0 and ib % nbuf == 0

    @functools.partial(
        pl.kernel,
        out_type=jax.ShapeDtypeStruct((NC, P, H), jnp.float32),
        mesh=_sc_mesh(),
        scratch_types=[
            pltpu.VMEM_SHARED((P, H), jnp.float32),
            pltpu.VMEM((ib, K), jnp.int32),
            pltpu.VMEM((ib, K), jnp.int32),
            [pltpu.VMEM((K, H), jnp.float32)] * nbuf,
            [pltpu.SemaphoreType.DMA] * nbuf,
        ],
    )
    def agg(z_hbm, src_hbm, dst_hbm, zero_hbm, out_hbm,
            acc, src_v, dst_v, rows, gsem):
        cid = lax.axis_index("c")
        sid = lax.axis_index("s")
        wid = cid * NS + sid
        row0 = sid * RPS
        pltpu.sync_copy(zero_hbm, acc.at[pl.ds(row0, RPS)])
        plsc.subcore_barrier()

        def group(c0):
            gs = [pltpu.async_copy(z_hbm.at[src_v.at[c0 + j]], rows[j],
                                   gsem[j]) for j in range(nbuf)]
            for j in range(nbuf):
                gs[j].wait()
                pltpu.sync_copy(rows[j], acc.at[dst_v.at[c0 + j]], add=True)

        def block(b, carry):
            pltpu.sync_copy(src_hbm.at[pl.ds(wid * cpw + b * ib, ib)], src_v)
            pltpu.sync_copy(dst_hbm.at[pl.ds(wid * cpw + b * ib, ib)], dst_v)

            def body(i, carry2):
                group(i * nbuf)
                return carry2

            return lax.fori_loop(0, ib // nbuf, body, carry)

        lax.fori_loop(0, nb, block, 0)
        plsc.subcore_barrier()
        pltpu.sync_copy(acc.at[pl.ds(row0, RPS)],
                        out_hbm.at[cid, pl.ds(row0, RPS)])

    return agg


_agg_main = _make_agg(CM)
_agg_meta = _make_agg(CMM)


def _entry_body(x_ref, wl_ref, bl_ref, mx_ref, wm_ref, bm_ref, o1_ref, o2_ref):
    o1_ref[...] = _lrelu(
        jnp.dot(x_ref[...], wl_ref[...], preferred_element_type=jnp.float32)
        + bl_ref[...])
    o2_ref[...] = _lrelu(
        jnp.dot(mx_ref[...], wm_ref[...], preferred_element_type=jnp.float32)
        + bm_ref[...])


def _dinv(deg):
    return lax.rsqrt(deg[0, :, 0:1] + deg[1, :, 0:1] + 1.0)


def _scale_body(h_ref, w_ref, deg_ref, z_ref):
    z_ref[...] = _dinv(deg_ref[...]) * jnp.dot(
        h_ref[...], w_ref[...], preferred_element_type=jnp.float32)


def _mid_body(agg_ref, z_ref, deg_ref, b_ref, w_ref, o_ref):
    dinv = _dinv(deg_ref[...])
    h = _lrelu(dinv * (agg_ref[0] + agg_ref[1] + z_ref[...]) + b_ref[...])
    o_ref[...] = dinv * jnp.dot(h, w_ref[...],
                                preferred_element_type=jnp.float32)


def _meta_body(agg_ref, z_ref, deg_ref, b_ref, mx_ref, w_ref, mdeg_ref, o_ref):
    dinv = _dinv(deg_ref[...])
    h = _lrelu(dinv * (agg_ref[0] + agg_ref[1] + z_ref[...]) + b_ref[...])
    h = jnp.concatenate([h[:N], mx_ref[...], h[N + 8:]], axis=0)
    o_ref[...] = _dinv(mdeg_ref[...]) * jnp.dot(
        h, w_ref[...], preferred_element_type=jnp.float32)


def _final_body(agg_ref, z_ref, mdeg_ref, b_ref, wc_ref, bc_ref, o_ref):
    dinv = _dinv(mdeg_ref[...])
    xm = _lrelu(dinv * (agg_ref[0] + agg_ref[1] + z_ref[...]) + b_ref[...])
    logits = jnp.dot(xm, wc_ref[...],
                     preferred_element_type=jnp.float32) + bc_ref[...]
    mx = jnp.max(logits, axis=1, keepdims=True)
    lse = jnp.log(jnp.sum(jnp.exp(logits - mx), axis=1, keepdims=True))
    o_ref[...] = logits - mx - lse


def _tc(body, out_shape, *args):
    return pl.pallas_call(body, out_shape=out_shape)(*args)


def kernel(x, edge_index, meta_edge_index, meta_x, W_lin, b_lin, W_mlin,
           b_mlin, W_conv0, b_conv0, W_conv1, b_conv1, W_mg, b_mg,
           W_cls, b_cls):
    f32 = jnp.float32
    x_pad = jnp.zeros((P, D), f32).at[:N].set(x)
    mx_pad = jnp.zeros((8, D), f32).at[:M].set(meta_x)
    src = jnp.full((EP,), P - 1, jnp.int32).at[:E].set(
        edge_index[0].astype(jnp.int32)).reshape(EP // K, K)
    dst = jnp.full((EP,), P - 1, jnp.int32).at[:E].set(
        edge_index[1].astype(jnp.int32)).reshape(EP // K, K)
    msrc = jnp.full((EMP,), P - 1, jnp.int32).at[:EM].set(
        meta_edge_index[0].astype(jnp.int32)).reshape(EMP // K, K)
    mdst = jnp.full((EMP,), P - 1, jnp.int32).at[:EM].set(
        meta_edge_index[1].astype(jnp.int32)).reshape(EMP // K, K)
    zero128 = jnp.zeros((RPS, H), f32)
    zero8 = jnp.zeros((RPS, GW), f32)
    ones8 = jnp.ones((K, GW), f32)
    wcls_pad = jnp.zeros((H, 128), f32).at[:, :C].set(W_cls)
    bcls_pad = jnp.full((1, 128), NEG, f32).at[0, :C].set(b_cls)

    degp, mdegp = _sc_degrees(dst, mdst, zero8, ones8)

    sds = jax.ShapeDtypeStruct
    x1, mx1 = _tc(_entry_body, (sds((P, H), f32), sds((8, H), f32)),
                  x_pad, W_lin, b_lin.reshape(1, H),
                  mx_pad, W_mlin, b_mlin.reshape(1, H))

    z0 = _tc(_scale_body, sds((P, H), f32), x1, W_conv0, degp)
    agg0 = _agg_main(z0, src, dst, zero128)
    z1 = _tc(_mid_body, sds((P, H), f32),
             agg0, z0, degp, b_conv0.reshape(1, H), W_conv1)
    agg1 = _agg_main(z1, src, dst, zero128)
    z2 = _tc(_meta_body, sds((P, H), f32),
             agg1, z1, degp, b_conv1.reshape(1, H), mx1, W_mg, mdegp)
    magg = _agg_meta(z2, msrc, mdst, zero128)
    outp = _tc(_final_body, sds((P, 128), f32),
               magg, z2, mdegp, b_mg.reshape(1, H), wcls_pad, bcls_pad)
    return outp[:N + M, :C]


# speedup vs baseline: 8.5551x; 1.0001x over previous
import functools

import jax
import jax.numpy as jnp
from jax import lax
from jax.experimental import pallas as pl
from jax.experimental.pallas import tpu as pltpu
from jax.experimental.pallas import tpu_sc as plsc

N = 10000
M = 3
D = 128
H = 128
C = 40
E = 320000
EM = 20000
P = 10240
NC, NS = 2, 16
NW = NC * NS
K = 80
EP = 327680
CM = EP // (NW * K)
EMP = 20480
CMM = EMP // (NW * K)
RPS = P // NS
ALPHA = 0.2
NEG = -1e30
GW = 8


def _lrelu(v):
    return jnp.where(v >= 0, v, ALPHA * v)


def _sc_mesh():
    return plsc.VectorSubcoreMesh(core_axis_name="c", subcore_axis_name="s")


@functools.partial(
    pl.kernel,
    out_type=(
        jax.ShapeDtypeStruct((NC, P, GW), jnp.float32),
        jax.ShapeDtypeStruct((NC, P, GW), jnp.float32),
    ),
    mesh=_sc_mesh(),
    scratch_types=[
        pltpu.VMEM_SHARED((P, GW), jnp.float32),
        pltpu.VMEM_SHARED((P, GW), jnp.float32),
        pltpu.VMEM((CM, K), jnp.int32),
        pltpu.VMEM((CMM, K), jnp.int32),
        pltpu.VMEM((K, GW), jnp.float32),
    ],
)
def _sc_degrees(dst_hbm, mdst_hbm, zero8_hbm, ones8_hbm,
                outd_hbm, outmd_hbm, accd, accmd, dst_v, mdst_v, ones_v):
    cid = lax.axis_index("c")
    sid = lax.axis_index("s")
    wid = cid * NS + sid
    row0 = sid * RPS
    pltpu.sync_copy(zero8_hbm, accd.at[pl.ds(row0, RPS)])
    pltpu.sync_copy(zero8_hbm, accmd.at[pl.ds(row0, RPS)])
    pltpu.sync_copy(ones8_hbm, ones_v)
    pltpu.sync_copy(dst_hbm.at[pl.ds(wid * CM, CM)], dst_v)
    pltpu.sync_copy(mdst_hbm.at[pl.ds(wid * CMM, CMM)], mdst_v)
    plsc.subcore_barrier()

    def body(c, carry):
        pltpu.sync_copy(ones_v, accd.at[dst_v.at[c]], add=True)
        return carry

    lax.fori_loop(0, CM, body, 0)

    def mbody(c, carry):
        pltpu.sync_copy(ones_v, accmd.at[mdst_v.at[c]], add=True)
        return carry

    lax.fori_loop(0, CMM, mbody, 0)
    plsc.subcore_barrier()
    pltpu.sync_copy(accd.at[pl.ds(row0, RPS)], outd_hbm.at[cid, pl.ds(row0, RPS)])
    pltpu.sync_copy(accmd.at[pl.ds(row0, RPS)], outmd_hbm.at[cid, pl.ds(row0, RPS)])


def _make_agg(cpw):
    ib = min(16, cpw)
    nb = cpw // ib
    nbuf = 4 if cpw % 4 == 0 else 2
    assert cpw % ib == 0 and ib % nbuf == 0

    @functools.partial(
        pl.kernel,
        out_type=jax.ShapeDtypeStruct((NC, P, H), jnp.float32),
        mesh=_sc_mesh(),
        scratch_types=(
            [pltpu.VMEM_SHARED((P, H), jnp.float32),
             pltpu.VMEM((ib, K), jnp.int32),
             pltpu.VMEM((ib, K), jnp.int32)]
            + [pltpu.VMEM((K, H), jnp.float32) for _ in range(nbuf)]
            + [pltpu.SemaphoreType.DMA for _ in range(nbuf)]
        ),
    )
    def agg(z_hbm, src_hbm, dst_hbm, zero_hbm, out_hbm,
            acc, src_v, dst_v, *rest):
        rows = rest[:nbuf]
        gsem = rest[nbuf:]
        cid = lax.axis_index("c")
        sid = lax.axis_index("s")
        wid = cid * NS + sid
        row0 = sid * RPS
        pltpu.sync_copy(zero_hbm, acc.at[pl.ds(row0, RPS)])
        plsc.subcore_barrier()

        def group(c0):
            gs = [pltpu.async_copy(z_hbm.at[src_v.at[c0 + j]], rows[j],
                                   gsem[j]) for j in range(nbuf)]
            for j in range(nbuf):
                gs[j].wait()
                pltpu.sync_copy(rows[j], acc.at[dst_v.at[c0 + j]], add=True)

        def block(b, carry):
            pltpu.sync_copy(src_hbm.at[pl.ds(wid * cpw + b * ib, ib)], src_v)
            pltpu.sync_copy(dst_hbm.at[pl.ds(wid * cpw + b * ib, ib)], dst_v)

            def body(i, carry2):
                group(i * nbuf)
                return carry2

            return lax.fori_loop(0, ib // nbuf, body, carry)

        lax.fori_loop(0, nb, block, 0)
        plsc.subcore_barrier()
        pltpu.sync_copy(acc.at[pl.ds(row0, RPS)],
                        out_hbm.at[cid, pl.ds(row0, RPS)])

    return agg


_agg_main = _make_agg(CM)
_agg_meta = _make_agg(CMM)


def _entry_body(x_ref, wl_ref, bl_ref, mx_ref, wm_ref, bm_ref, o1_ref, o2_ref):
    o1_ref[...] = _lrelu(
        jnp.dot(x_ref[...], wl_ref[...], preferred_element_type=jnp.float32)
        + bl_ref[...])
    o2_ref[...] = _lrelu(
        jnp.dot(mx_ref[...], wm_ref[...], preferred_element_type=jnp.float32)
        + bm_ref[...])


def _dinv(deg):
    return lax.rsqrt(deg[0, :, 0:1] + deg[1, :, 0:1] + 1.0)


def _scale_body(h_ref, w_ref, deg_ref, z_ref):
    z_ref[...] = _dinv(deg_ref[...]) * jnp.dot(
        h_ref[...], w_ref[...], preferred_element_type=jnp.float32)


def _mid_body(agg_ref, z_ref, deg_ref, b_ref, w_ref, o_ref):
    dinv = _dinv(deg_ref[...])
    h = _lrelu(dinv * (agg_ref[0] + agg_ref[1] + z_ref[...]) + b_ref[...])
    o_ref[...] = dinv * jnp.dot(h, w_ref[...],
                                preferred_element_type=jnp.float32)


def _meta_body(agg_ref, z_ref, deg_ref, b_ref, mx_ref, w_ref, mdeg_ref, o_ref):
    dinv = _dinv(deg_ref[...])
    h = _lrelu(dinv * (agg_ref[0] + agg_ref[1] + z_ref[...]) + b_ref[...])
    h = jnp.concatenate([h[:N], mx_ref[...], h[N + 8:]], axis=0)
    o_ref[...] = _dinv(mdeg_ref[...]) * jnp.dot(
        h, w_ref[...], preferred_element_type=jnp.float32)


def _final_body(agg_ref, z_ref, mdeg_ref, b_ref, wc_ref, bc_ref, o_ref):
    dinv = _dinv(mdeg_ref[...])
    xm = _lrelu(dinv * (agg_ref[0] + agg_ref[1] + z_ref[...]) + b_ref[...])
    logits = jnp.dot(xm, wc_ref[...],
                     preferred_element_type=jnp.float32) + bc_ref[...]
    mx = jnp.max(logits, axis=1, keepdims=True)
    lse = jnp.log(jnp.sum(jnp.exp(logits - mx), axis=1, keepdims=True))
    o_ref[...] = logits - mx - lse


def _tc(body, out_shape, *args):
    return pl.pallas_call(body, out_shape=out_shape)(*args)


def kernel(x, edge_index, meta_edge_index, meta_x, W_lin, b_lin, W_mlin,
           b_mlin, W_conv0, b_conv0, W_conv1, b_conv1, W_mg, b_mg,
           W_cls, b_cls):
    f32 = jnp.float32
    x_pad = jnp.zeros((P, D), f32).at[:N].set(x)
    mx_pad = jnp.zeros((8, D), f32).at[:M].set(meta_x)
    src = jnp.full((EP,), P - 1, jnp.int32).at[:E].set(
        edge_index[0].astype(jnp.int32)).reshape(EP // K, K)
    dst = jnp.full((EP,), P - 1, jnp.int32).at[:E].set(
        edge_index[1].astype(jnp.int32)).reshape(EP // K, K)
    msrc = jnp.full((EMP,), P - 1, jnp.int32).at[:EM].set(
        meta_edge_index[0].astype(jnp.int32)).reshape(EMP // K, K)
    mdst = jnp.full((EMP,), P - 1, jnp.int32).at[:EM].set(
        meta_edge_index[1].astype(jnp.int32)).reshape(EMP // K, K)
    zero128 = jnp.zeros((RPS, H), f32)
    zero8 = jnp.zeros((RPS, GW), f32)
    ones8 = jnp.ones((K, GW), f32)
    wcls_pad = jnp.zeros((H, 128), f32).at[:, :C].set(W_cls)
    bcls_pad = jnp.full((1, 128), NEG, f32).at[0, :C].set(b_cls)

    degp, mdegp = _sc_degrees(dst, mdst, zero8, ones8)

    sds = jax.ShapeDtypeStruct
    x1, mx1 = _tc(_entry_body, (sds((P, H), f32), sds((8, H), f32)),
                  x_pad, W_lin, b_lin.reshape(1, H),
                  mx_pad, W_mlin, b_mlin.reshape(1, H))

    z0 = _tc(_scale_body, sds((P, H), f32), x1, W_conv0, degp)
    agg0 = _agg_main(z0, src, dst, zero128)
    z1 = _tc(_mid_body, sds((P, H), f32),
             agg0, z0, degp, b_conv0.reshape(1, H), W_conv1)
    agg1 = _agg_main(z1, src, dst, zero128)
    z2 = _tc(_meta_body, sds((P, H), f32),
             agg1, z1, degp, b_conv1.reshape(1, H), mx1, W_mg, mdegp)
    magg = _agg_meta(z2, msrc, mdst, zero128)
    outp = _tc(_final_body, sds((P, 128), f32),
               magg, z2, mdegp, b_mg.reshape(1, H), wcls_pad, bcls_pad)
    return outp[:N + M, :C]


# speedup vs baseline: 8.6667x; 1.0130x over previous
import functools

import jax
import jax.numpy as jnp
from jax import lax
from jax.experimental import pallas as pl
from jax.experimental.pallas import tpu as pltpu
from jax.experimental.pallas import tpu_sc as plsc

N = 10000
M = 3
D = 128
H = 128
C = 40
E = 320000
EM = 20000
P = 10240
NC, NS = 2, 16
NW = NC * NS
K = 80
EP = 327680
CM = EP // (NW * K)
EMP = 20480
CMM = EMP // (NW * K)
RPS = P // NS
ALPHA = 0.2
NEG = -1e30
GW = 8


def _lrelu(v):
    return jnp.where(v >= 0, v, ALPHA * v)


def _sc_mesh():
    return plsc.VectorSubcoreMesh(core_axis_name="c", subcore_axis_name="s")


@functools.partial(
    pl.kernel,
    out_type=(
        jax.ShapeDtypeStruct((NC, P, GW), jnp.float32),
        jax.ShapeDtypeStruct((NC, P, GW), jnp.float32),
    ),
    mesh=_sc_mesh(),
    scratch_types=[
        pltpu.VMEM_SHARED((P, GW), jnp.float32),
        pltpu.VMEM_SHARED((P, GW), jnp.float32),
        pltpu.VMEM((CM, K), jnp.int32),
        pltpu.VMEM((CMM, K), jnp.int32),
        pltpu.VMEM((K, GW), jnp.float32),
    ],
)
def _sc_degrees(dst_hbm, mdst_hbm, zero8_hbm, ones8_hbm,
                outd_hbm, outmd_hbm, accd, accmd, dst_v, mdst_v, ones_v):
    cid = lax.axis_index("c")
    sid = lax.axis_index("s")
    wid = cid * NS + sid
    row0 = sid * RPS
    pltpu.sync_copy(zero8_hbm, accd.at[pl.ds(row0, RPS)])
    pltpu.sync_copy(zero8_hbm, accmd.at[pl.ds(row0, RPS)])
    pltpu.sync_copy(ones8_hbm, ones_v)
    pltpu.sync_copy(dst_hbm.at[pl.ds(wid * CM, CM)], dst_v)
    pltpu.sync_copy(mdst_hbm.at[pl.ds(wid * CMM, CMM)], mdst_v)
    plsc.subcore_barrier()

    def body(c, carry):
        pltpu.sync_copy(ones_v, accd.at[dst_v.at[c]], add=True)
        return carry

    lax.fori_loop(0, CM, body, 0)

    def mbody(c, carry):
        pltpu.sync_copy(ones_v, accmd.at[mdst_v.at[c]], add=True)
        return carry

    lax.fori_loop(0, CMM, mbody, 0)
    plsc.subcore_barrier()
    pltpu.sync_copy(accd.at[pl.ds(row0, RPS)], outd_hbm.at[cid, pl.ds(row0, RPS)])
    pltpu.sync_copy(accmd.at[pl.ds(row0, RPS)], outmd_hbm.at[cid, pl.ds(row0, RPS)])


def _make_agg(cpw, k):
    ib = 16 if cpw % 16 == 0 else 8
    nbuf = 2 if k > 96 else 4
    assert cpw % ib == 0 and ib % nbuf == 0 and cpw % 8 == 0

    @functools.partial(
        pl.kernel,
        out_type=jax.ShapeDtypeStruct((NC, P, H), jnp.float32),
        mesh=_sc_mesh(),
        scratch_types=(
            [pltpu.VMEM_SHARED((P, H), jnp.float32),
             pltpu.VMEM((ib, k), jnp.int32),
             pltpu.VMEM((ib, k), jnp.int32)]
            + [pltpu.VMEM((k, H), jnp.float32) for _ in range(nbuf)]
            + [pltpu.SemaphoreType.DMA for _ in range(nbuf)]
        ),
    )
    def agg(z_hbm, src_hbm, dst_hbm, zero_hbm, out_hbm,
            acc, src_v, dst_v, *rest):
        rows = rest[:nbuf]
        gsem = rest[nbuf:]
        cid = lax.axis_index("c")
        sid = lax.axis_index("s")
        row0 = sid * RPS
        base = (cid * NS + sid) * cpw
        pltpu.sync_copy(zero_hbm, acc.at[pl.ds(row0, RPS)])
        plsc.subcore_barrier()

        def group(c0):
            gs = [pltpu.async_copy(z_hbm.at[src_v.at[c0 + j]], rows[j],
                                   gsem[j]) for j in range(nbuf)]
            for j in range(nbuf):
                gs[j].wait()
                pltpu.sync_copy(rows[j], acc.at[dst_v.at[c0 + j]], add=True)

        def block(b, carry):
            pltpu.sync_copy(src_hbm.at[pl.ds(base + b * ib, ib)], src_v)
            pltpu.sync_copy(dst_hbm.at[pl.ds(base + b * ib, ib)], dst_v)

            def body(i, carry2):
                group(i * nbuf)
                return carry2

            return lax.fori_loop(0, ib // nbuf, body, carry)

        lax.fori_loop(0, cpw // ib, block, 0)
        plsc.subcore_barrier()
        pltpu.sync_copy(acc.at[pl.ds(row0, RPS)],
                        out_hbm.at[cid, pl.ds(row0, RPS)])

    return agg


KA = 128
CA = EP // (NW * KA)
_agg_main = _make_agg(CA, KA)
_agg_meta = _make_agg(CMM, K)


def _entry_body(x_ref, wl_ref, bl_ref, mx_ref, wm_ref, bm_ref, o1_ref, o2_ref):
    o1_ref[...] = _lrelu(
        jnp.dot(x_ref[...], wl_ref[...], preferred_element_type=jnp.float32)
        + bl_ref[...])
    o2_ref[...] = _lrelu(
        jnp.dot(mx_ref[...], wm_ref[...], preferred_element_type=jnp.float32)
        + bm_ref[...])


def _dinv(deg):
    return lax.rsqrt(deg[0, :, 0:1] + deg[1, :, 0:1] + 1.0)


def _scale_body(h_ref, w_ref, deg_ref, z_ref):
    z_ref[...] = _dinv(deg_ref[...]) * jnp.dot(
        h_ref[...], w_ref[...], preferred_element_type=jnp.float32)


def _mid_body(agg_ref, z_ref, deg_ref, b_ref, w_ref, o_ref):
    dinv = _dinv(deg_ref[...])
    h = _lrelu(dinv * (agg_ref[0] + agg_ref[1] + z_ref[...]) + b_ref[...])
    o_ref[...] = dinv * jnp.dot(h, w_ref[...],
                                preferred_element_type=jnp.float32)


def _meta_body(agg_ref, z_ref, deg_ref, b_ref, mx_ref, w_ref, mdeg_ref, o_ref):
    dinv = _dinv(deg_ref[...])
    h = _lrelu(dinv * (agg_ref[0] + agg_ref[1] + z_ref[...]) + b_ref[...])
    h = jnp.concatenate([h[:N], mx_ref[...], h[N + 8:]], axis=0)
    o_ref[...] = _dinv(mdeg_ref[...]) * jnp.dot(
        h, w_ref[...], preferred_element_type=jnp.float32)


def _final_body(agg_ref, z_ref, mdeg_ref, b_ref, wc_ref, bc_ref, o_ref):
    dinv = _dinv(mdeg_ref[...])
    xm = _lrelu(dinv * (agg_ref[0] + agg_ref[1] + z_ref[...]) + b_ref[...])
    logits = jnp.dot(xm, wc_ref[...],
                     preferred_element_type=jnp.float32) + bc_ref[...]
    mx = jnp.max(logits, axis=1, keepdims=True)
    lse = jnp.log(jnp.sum(jnp.exp(logits - mx), axis=1, keepdims=True))
    o_ref[...] = logits - mx - lse


def _tc(body, out_shape, *args):
    return pl.pallas_call(body, out_shape=out_shape)(*args)


def kernel(x, edge_index, meta_edge_index, meta_x, W_lin, b_lin, W_mlin,
           b_mlin, W_conv0, b_conv0, W_conv1, b_conv1, W_mg, b_mg,
           W_cls, b_cls):
    f32 = jnp.float32
    x_pad = jnp.zeros((P, D), f32).at[:N].set(x)
    mx_pad = jnp.zeros((8, D), f32).at[:M].set(meta_x)
    src_flat = jnp.full((EP,), P - 1, jnp.int32).at[:E].set(
        edge_index[0].astype(jnp.int32))
    dst_flat = jnp.full((EP,), P - 1, jnp.int32).at[:E].set(
        edge_index[1].astype(jnp.int32))
    srcA = src_flat.reshape(EP // KA, KA)
    dstA = dst_flat.reshape(EP // KA, KA)
    dst = dst_flat.reshape(EP // K, K)
    msrc = jnp.full((EMP,), P - 1, jnp.int32).at[:EM].set(
        meta_edge_index[0].astype(jnp.int32)).reshape(EMP // K, K)
    mdst = jnp.full((EMP,), P - 1, jnp.int32).at[:EM].set(
        meta_edge_index[1].astype(jnp.int32)).reshape(EMP // K, K)
    zero128 = jnp.zeros((RPS, H), f32)
    zero8 = jnp.zeros((RPS, GW), f32)
    ones8 = jnp.ones((K, GW), f32)
    wcls_pad = jnp.zeros((H, 128), f32).at[:, :C].set(W_cls)
    bcls_pad = jnp.full((1, 128), NEG, f32).at[0, :C].set(b_cls)

    degp, mdegp = _sc_degrees(dst, mdst, zero8, ones8)

    sds = jax.ShapeDtypeStruct
    x1, mx1 = _tc(_entry_body, (sds((P, H), f32), sds((8, H), f32)),
                  x_pad, W_lin, b_lin.reshape(1, H),
                  mx_pad, W_mlin, b_mlin.reshape(1, H))

    z0 = _tc(_scale_body, sds((P, H), f32), x1, W_conv0, degp)
    agg0 = _agg_main(z0, srcA, dstA, zero128)
    z1 = _tc(_mid_body, sds((P, H), f32),
             agg0, z0, degp, b_conv0.reshape(1, H), W_conv1)
    agg1 = _agg_main(z1, srcA, dstA, zero128)
    z2 = _tc(_meta_body, sds((P, H), f32),
             agg1, z1, degp, b_conv1.reshape(1, H), mx1, W_mg, mdegp)
    magg = _agg_meta(z2, msrc, mdst, zero128)
    outp = _tc(_final_body, sds((P, 128), f32),
               magg, z2, mdegp, b_mg.reshape(1, H), wcls_pad, bcls_pad)
    return outp[:N + M, :C]
